# Initial kernel scaffold; baseline (speedup 1.0000x reference)
import jax
import jax.numpy as jnp
from jax.experimental import pallas as pl


def kernel(S, table):
    raise NotImplementedError("write your pallas kernel here")


# speedup vs baseline: 1.2218x; 1.2218x over previous
import functools

import jax
import jax.numpy as jnp
from jax import lax
from jax.experimental import pallas as pl
from jax.experimental.pallas import tpu as pltpu
from jax.experimental.pallas import tpu_sc as plsc

_EMB_DIM = 64
_B = 4096
_BASKET = 20
_NC = 2
_NS = 16
_NW = _NC * _NS
_BPW = _B // _NW
_U = 4
_ROWS = _U * _BASKET
_STEPS = _BPW // _U
_LANES = 16
_DCOLS = _EMB_DIM // _LANES

_mesh = plsc.VectorSubcoreMesh(core_axis_name="c", subcore_axis_name="s")


@functools.partial(
    pl.kernel,
    mesh=_mesh,
    out_type=jax.ShapeDtypeStruct((_B, _EMB_DIM), jnp.float32),
    compiler_params=pltpu.CompilerParams(use_tc_tiling_on_sc=False),
    scratch_types=[
        pltpu.VMEM((_BPW * _BASKET,), jnp.int32),
        pltpu.VMEM((_ROWS, _EMB_DIM), jnp.float32),
        pltpu.VMEM((_U, _EMB_DIM), jnp.float32),
        pltpu.SemaphoreType.DMA,
    ],
)
def _basket_sum(idx_hbm, table_hbm, out_hbm, idx_v, rows_v, out_v, sem):
    wid = lax.axis_index("s") * _NC + lax.axis_index("c")
    ibase = wid * (_BPW * _BASKET)
    ubase = wid * _BPW
    pltpu.sync_copy(idx_hbm.at[pl.ds(ibase, _BPW * _BASKET)], idx_v)

    def step(s, carry):
        pltpu.async_copy(
            table_hbm.at[idx_v.at[pl.ds(s * _ROWS, _ROWS)]], rows_v, sem
        ).wait()
        for u in range(_U):
            for d in range(_DCOLS):
                acc = rows_v[u * _BASKET, pl.ds(d * _LANES, _LANES)]
                for j in range(1, _BASKET):
                    acc = acc + rows_v[u * _BASKET + j, pl.ds(d * _LANES, _LANES)]
                out_v[u, pl.ds(d * _LANES, _LANES)] = acc
        pltpu.sync_copy(out_v, out_hbm.at[pl.ds(ubase + s * _U, _U)])
        return carry

    lax.fori_loop(0, _STEPS, step, 0)


def kernel(S, table):
    idx = S[:, -1, :].astype(jnp.int32).reshape(_B * _BASKET)
    return _basket_sum(idx, table)


# speedup vs baseline: 1.2228x; 1.0008x over previous
import functools

import jax
import jax.numpy as jnp
from jax import lax
from jax.experimental import pallas as pl
from jax.experimental.pallas import tpu as pltpu
from jax.experimental.pallas import tpu_sc as plsc

_EMB_DIM = 64
_B = 4096
_BASKET = 20
_NC = 2
_NS = 16
_NW = _NC * _NS
_BPW = _B // _NW
_U = 4
_ROWS = _U * _BASKET
_STEPS = _BPW // _U
_NBUF = 4
_LANES = 16
_DCOLS = _EMB_DIM // _LANES

_mesh = plsc.VectorSubcoreMesh(core_axis_name="c", subcore_axis_name="s")


@functools.partial(
    pl.kernel,
    mesh=_mesh,
    out_type=jax.ShapeDtypeStruct((_B, _EMB_DIM), jnp.float32),
    compiler_params=pltpu.CompilerParams(use_tc_tiling_on_sc=False),
    scratch_types=[
        pltpu.VMEM((_BPW * _BASKET,), jnp.int32),
        pltpu.VMEM((_NBUF, _ROWS, _EMB_DIM), jnp.float32),
        pltpu.VMEM((_BPW, _EMB_DIM), jnp.float32),
        [pltpu.SemaphoreType.DMA] * _NBUF,
    ],
)
def _basket_sum(idx_hbm, table_hbm, out_hbm, idx_v, rows_v, out_v, sems):
    wid = lax.axis_index("s") * _NC + lax.axis_index("c")
    ibase = wid * (_BPW * _BASKET)
    ubase = wid * _BPW
    pltpu.sync_copy(idx_hbm.at[pl.ds(ibase, _BPW * _BASKET)], idx_v)

    def gather(s, b):
        return pltpu.make_async_copy(
            table_hbm.at[idx_v.at[pl.ds(s * _ROWS, _ROWS)]], rows_v.at[b], sems[b]
        )

    for b in range(_NBUF):
        gather(b, b).start()

    def outer(g, carry):
        for b in range(_NBUF):
            s = g * _NBUF + b
            gather(s, b).wait()
            for u in range(_U):
                for d in range(_DCOLS):
                    acc = rows_v[b, u * _BASKET, pl.ds(d * _LANES, _LANES)]
                    for j in range(1, _BASKET):
                        acc = acc + rows_v[
                            b, u * _BASKET + j, pl.ds(d * _LANES, _LANES)
                        ]
                    out_v[s * _U + u, pl.ds(d * _LANES, _LANES)] = acc

            @pl.when(s + _NBUF < _STEPS)
            def _():
                gather(s + _NBUF, b).start()

        return carry

    lax.fori_loop(0, _STEPS // _NBUF, outer, 0)
    pltpu.sync_copy(out_v, out_hbm.at[pl.ds(ubase, _BPW)])


def kernel(S, table):
    idx = S[:, -1, :].astype(jnp.int32).reshape(_B * _BASKET)
    return _basket_sum(idx, table)
